# Initial kernel scaffold; baseline (speedup 1.0000x reference)
import jax
import jax.numpy as jnp
from jax.experimental import pallas as pl


def kernel(x0_word_ids, x1_char_feats, table, W2, b2, W3, b3, W4, b4):
    raise NotImplementedError("write your pallas kernel here")


# speedup vs baseline: 1.5913x; 1.5913x over previous
import functools

import jax
import jax.numpy as jnp
from jax import lax
from jax.experimental import pallas as pl
from jax.experimental.pallas import tpu as pltpu
from jax.experimental.pallas import tpu_sc as plsc

D = 128
CIN = 32
COUT = 64
WLEN = 16
NOUT = 3 * COUT
XW = WLEN * CIN
BT = 512


def _conv_body(x_ref, emb_ref, w_ref, b_ref, char_ref, fwo_ref):
    x = x_ref[...]
    xp = jnp.concatenate([x, jnp.zeros((x.shape[0], 2 * CIN), jnp.float32)], axis=1)
    w = w_ref[...]
    b = b_ref[...]
    col = lax.broadcasted_iota(jnp.int32, (x.shape[0], NOUT), 1)
    acc = None
    for p in range(WLEN - 1):
        h = jnp.tanh(
            jnp.dot(xp[:, CIN * p:CIN * p + 4 * CIN], w,
                    preferred_element_type=jnp.float32) + b
        )
        if p == WLEN - 3:
            h = jnp.where(col < 2 * COUT, h, -jnp.inf)
        elif p == WLEN - 2:
            h = jnp.where(col < COUT, h, -jnp.inf)
        acc = h if acc is None else jnp.maximum(acc, h)
    char_ref[...] = acc
    fwo_ref[...] = jnp.concatenate([acc, emb_ref[...]], axis=1)


def _conv_call(x_flat, emb_flat, wbig, bcat):
    t = x_flat.shape[0]
    grid = t // BT
    return pl.pallas_call(
        _conv_body,
        grid=(grid,),
        in_specs=[
            pl.BlockSpec((BT, XW), lambda i: (i, 0)),
            pl.BlockSpec((BT, D), lambda i: (i, 0)),
            pl.BlockSpec((4 * CIN, NOUT), lambda i: (0, 0)),
            pl.BlockSpec((1, NOUT), lambda i: (0, 0)),
        ],
        out_specs=[
            pl.BlockSpec((BT, NOUT), lambda i: (i, 0)),
            pl.BlockSpec((BT, NOUT + D), lambda i: (i, 0)),
        ],
        out_shape=[
            jax.ShapeDtypeStruct((t, NOUT), jnp.float32),
            jax.ShapeDtypeStruct((t, NOUT + D), jnp.float32),
        ],
    )(x_flat, emb_flat, wbig, bcat)


@functools.cache
def _make_gather(vocab, d, t):
    info = plsc.get_sparse_core_info()
    nw = info.num_cores * info.num_subcores
    t_per_w = t // nw
    ch = 400
    n_ch = t_per_w // ch
    mesh = plsc.VectorSubcoreMesh(core_axis_name="c", subcore_axis_name="s")

    @functools.partial(
        pl.kernel,
        mesh=mesh,
        out_type=jax.ShapeDtypeStruct((t, d), jnp.float32),
        scratch_types=[
            pltpu.VMEM((t_per_w,), jnp.int32),
            pltpu.VMEM((ch, d), jnp.float32),
            pltpu.SemaphoreType.DMA,
        ],
    )
    def gather_k(idx_hbm, table_hbm, out_hbm, idx_v, rows_v, sem):
        wid = lax.axis_index("s") * info.num_cores + lax.axis_index("c")
        base = wid * t_per_w
        pltpu.sync_copy(idx_hbm.at[pl.ds(base, t_per_w)], idx_v)
        for c in range(n_ch):
            pltpu.async_copy(
                table_hbm.at[idx_v.at[pl.ds(c * ch, ch)]], rows_v, sem
            ).wait()
            pltpu.sync_copy(rows_v, out_hbm.at[pl.ds(base + c * ch, ch)])

    return gather_k


def kernel(x0_word_ids, x1_char_feats, table, W2, b2, W3, b3, W4, b4):
    b, l = x0_word_ids.shape
    t = b * l
    idx = x0_word_ids.reshape(t).astype(jnp.int32)
    emb_flat = _make_gather(table.shape[0], D, t)(idx, table)

    w2p = jnp.pad(W2, ((0, 2 * CIN), (0, 0)))
    w3p = jnp.pad(W3, ((0, CIN), (0, 0)))
    wbig = jnp.concatenate([w2p, w3p, W4], axis=1)
    bcat = jnp.concatenate([b2, b3, b4]).reshape(1, NOUT)

    x_flat = x1_char_feats.reshape(t, XW)
    char_flat, fwo_flat = _conv_call(x_flat, emb_flat, wbig, bcat)

    return (
        emb_flat.reshape(b, l, D),
        char_flat.reshape(b, l, NOUT),
        fwo_flat.reshape(b, l, NOUT + D),
    )


# speedup vs baseline: 7.9335x; 4.9855x over previous
import functools

import jax
import jax.numpy as jnp
from jax import lax
from jax.experimental import pallas as pl
from jax.experimental.pallas import tpu as pltpu
from jax.experimental.pallas import tpu_sc as plsc

D = 128
CIN = 32
COUT = 64
WLEN = 16
NOUT = 3 * COUT
XW = WLEN * CIN


def _conv_body(x_ref, emb_ref, w_ref, b_ref, char_ref, fwo_ref):
    x = x_ref[0]
    nb = x.shape[1]
    xp = jnp.concatenate([x, jnp.zeros((2 * CIN, nb), jnp.float32)], axis=0)
    w = w_ref[...]
    bb = b_ref[...]
    row = lax.broadcasted_iota(jnp.int32, (NOUT, nb), 0)
    acc = None
    for p in range(WLEN - 1):
        h = jnp.tanh(
            jnp.dot(w, xp[CIN * p:CIN * p + 4 * CIN, :],
                    preferred_element_type=jnp.float32) + bb
        )
        if p == WLEN - 3:
            h = jnp.where(row < 2 * COUT, h, -jnp.inf)
        elif p == WLEN - 2:
            h = jnp.where(row < COUT, h, -jnp.inf)
        acc = h if acc is None else jnp.maximum(acc, h)
    char_ref[0] = acc
    fwo_ref[0] = jnp.concatenate([acc, jnp.transpose(emb_ref[0])], axis=0)


def _conv_call(x_t, emb3, wbig_t, bcol):
    l, _, nb = x_t.shape
    return pl.pallas_call(
        _conv_body,
        grid=(l,),
        in_specs=[
            pl.BlockSpec((1, XW, nb), lambda i: (i, 0, 0)),
            pl.BlockSpec((1, nb, D), lambda i: (i, 0, 0)),
            pl.BlockSpec((NOUT, 4 * CIN), lambda i: (0, 0)),
            pl.BlockSpec((NOUT, 1), lambda i: (0, 0)),
        ],
        out_specs=[
            pl.BlockSpec((1, NOUT, nb), lambda i: (i, 0, 0)),
            pl.BlockSpec((1, NOUT + D, nb), lambda i: (i, 0, 0)),
        ],
        out_shape=[
            jax.ShapeDtypeStruct((l, NOUT, nb), jnp.float32),
            jax.ShapeDtypeStruct((l, NOUT + D, nb), jnp.float32),
        ],
    )(x_t, emb3, wbig_t, bcol)


@functools.cache
def _make_gather(vocab, d, t):
    info = plsc.get_sparse_core_info()
    nw = info.num_cores * info.num_subcores
    t_per_w = t // nw
    ch = 400
    n_ch = t_per_w // ch
    mesh = plsc.VectorSubcoreMesh(core_axis_name="c", subcore_axis_name="s")

    @functools.partial(
        pl.kernel,
        mesh=mesh,
        out_type=jax.ShapeDtypeStruct((t, d), jnp.float32),
        scratch_types=[
            pltpu.VMEM((t_per_w,), jnp.int32),
            pltpu.VMEM((ch, d), jnp.float32),
            pltpu.SemaphoreType.DMA,
        ],
    )
    def gather_k(idx_hbm, table_hbm, out_hbm, idx_v, rows_v, sem):
        wid = lax.axis_index("s") * info.num_cores + lax.axis_index("c")
        base = wid * t_per_w
        pltpu.sync_copy(idx_hbm.at[pl.ds(base, t_per_w)], idx_v)
        for c in range(n_ch):
            pltpu.async_copy(
                table_hbm.at[idx_v.at[pl.ds(c * ch, ch)]], rows_v, sem
            ).wait()
            pltpu.sync_copy(rows_v, out_hbm.at[pl.ds(base + c * ch, ch)])

    return gather_k


def kernel(x0_word_ids, x1_char_feats, table, W2, b2, W3, b3, W4, b4):
    b, l = x0_word_ids.shape
    t = b * l
    idx = jnp.transpose(x0_word_ids).reshape(t).astype(jnp.int32)
    emb_flat = _make_gather(table.shape[0], D, t)(idx, table)
    emb3 = emb_flat.reshape(l, b, D)

    w2p = jnp.pad(W2, ((0, 2 * CIN), (0, 0)))
    w3p = jnp.pad(W3, ((0, CIN), (0, 0)))
    wbig_t = jnp.transpose(jnp.concatenate([w2p, w3p, W4], axis=1))
    bcol = jnp.concatenate([b2, b3, b4]).reshape(NOUT, 1)

    x_t = jnp.transpose(x1_char_feats, (1, 2, 3, 0)).reshape(l, XW, b)
    char_t, fwo_t = _conv_call(x_t, emb3, wbig_t, bcol)

    return (
        jnp.transpose(emb3, (1, 0, 2)),
        jnp.transpose(char_t, (2, 0, 1)),
        jnp.transpose(fwo_t, (2, 0, 1)),
    )


# speedup vs baseline: 8.3188x; 1.0486x over previous
import functools

import jax
import jax.numpy as jnp
from jax import lax
from jax.experimental import pallas as pl
from jax.experimental.pallas import tpu as pltpu
from jax.experimental.pallas import tpu_sc as plsc

D = 128
CIN = 32
COUT = 64
WLEN = 16
NOUT = 3 * COUT
XW = WLEN * CIN


def _conv_body(x_ref, emb_ref, w_ref, b_ref, char_ref, fwo_ref):
    x = x_ref[0]
    nb = x.shape[1]
    w = w_ref[...]
    bb = b_ref[...]
    row = lax.broadcasted_iota(jnp.int32, (NOUT, nb), 0)
    acc = None
    for p in range(WLEN - 3):
        h = jnp.dot(w, x[CIN * p:CIN * p + 4 * CIN, :],
                    preferred_element_type=jnp.float32)
        acc = h if acc is None else jnp.maximum(acc, h)
    h = jnp.dot(w[:, :3 * CIN], x[CIN * (WLEN - 3):XW, :],
                preferred_element_type=jnp.float32)
    acc = jnp.where(row < 2 * COUT, jnp.maximum(acc, h), acc)
    h = jnp.dot(w[:, :2 * CIN], x[CIN * (WLEN - 2):XW, :],
                preferred_element_type=jnp.float32)
    acc = jnp.where(row < COUT, jnp.maximum(acc, h), acc)
    acc = jnp.tanh(acc + bb)
    char_ref[0] = acc
    fwo_ref[0] = jnp.concatenate([acc, jnp.transpose(emb_ref[0])], axis=0)


def _conv_call(x_t, emb3, wbig_t, bcol):
    l, _, nb = x_t.shape
    return pl.pallas_call(
        _conv_body,
        grid=(l,),
        in_specs=[
            pl.BlockSpec((1, XW, nb), lambda i: (i, 0, 0)),
            pl.BlockSpec((1, nb, D), lambda i: (i, 0, 0)),
            pl.BlockSpec((NOUT, 4 * CIN), lambda i: (0, 0)),
            pl.BlockSpec((NOUT, 1), lambda i: (0, 0)),
        ],
        out_specs=[
            pl.BlockSpec((1, NOUT, nb), lambda i: (i, 0, 0)),
            pl.BlockSpec((1, NOUT + D, nb), lambda i: (i, 0, 0)),
        ],
        out_shape=[
            jax.ShapeDtypeStruct((l, NOUT, nb), jnp.float32),
            jax.ShapeDtypeStruct((l, NOUT + D, nb), jnp.float32),
        ],
    )(x_t, emb3, wbig_t, bcol)


@functools.cache
def _make_gather(vocab, d, t):
    info = plsc.get_sparse_core_info()
    nw = info.num_cores * info.num_subcores
    t_per_w = t // nw
    ch = 400
    n_ch = t_per_w // ch
    mesh = plsc.VectorSubcoreMesh(core_axis_name="c", subcore_axis_name="s")

    @functools.partial(
        pl.kernel,
        mesh=mesh,
        out_type=jax.ShapeDtypeStruct((t, d), jnp.float32),
        scratch_types=[
            pltpu.VMEM((t_per_w,), jnp.int32),
            pltpu.VMEM((ch, d), jnp.float32),
            pltpu.SemaphoreType.DMA,
        ],
    )
    def gather_k(idx_hbm, table_hbm, out_hbm, idx_v, rows_v, sem):
        wid = lax.axis_index("s") * info.num_cores + lax.axis_index("c")
        base = wid * t_per_w
        pltpu.sync_copy(idx_hbm.at[pl.ds(base, t_per_w)], idx_v)
        for c in range(n_ch):
            pltpu.async_copy(
                table_hbm.at[idx_v.at[pl.ds(c * ch, ch)]], rows_v, sem
            ).wait()
            pltpu.sync_copy(rows_v, out_hbm.at[pl.ds(base + c * ch, ch)])

    return gather_k


def kernel(x0_word_ids, x1_char_feats, table, W2, b2, W3, b3, W4, b4):
    b, l = x0_word_ids.shape
    t = b * l
    idx = jnp.transpose(x0_word_ids).reshape(t).astype(jnp.int32)
    emb_flat = _make_gather(table.shape[0], D, t)(idx, table)
    emb3 = emb_flat.reshape(l, b, D)

    w2p = jnp.pad(W2, ((0, 2 * CIN), (0, 0)))
    w3p = jnp.pad(W3, ((0, CIN), (0, 0)))
    wbig_t = jnp.transpose(jnp.concatenate([w2p, w3p, W4], axis=1))
    bcol = jnp.concatenate([b2, b3, b4]).reshape(NOUT, 1)

    x_t = jnp.transpose(x1_char_feats, (1, 2, 3, 0)).reshape(l, XW, b)
    char_t, fwo_t = _conv_call(x_t, emb3, wbig_t, bcol)

    return (
        jnp.transpose(emb3, (1, 0, 2)),
        jnp.transpose(char_t, (2, 0, 1)),
        jnp.transpose(fwo_t, (2, 0, 1)),
    )
